# baseline (speedup 1.0000x reference)
import jax
import jax.numpy as jnp
from jax import lax
from jax.experimental import pallas as pl
from jax.experimental.pallas import tpu as pltpu
from jax.experimental.pallas import tpu_sc as plsc

NR_NODES = 1000000
EMB_DIM = 64
BATCH = 16384

SLOT_W = 2 * EMB_DIM
NSLOTS = NR_NODES // 2

NC = 2
NS = 16
L = 16
NW = NC * NS
BPW = BATCH // NW
CHUNK = 256
NCHUNK = BPW // CHUNK
CGROUPS = CHUNK // L


def _sc_body(tgt_hbm, ctx_hbm, wv_hbm, bv_hbm, table_hbm, out_hbm,
             idx_t, idx_c, slot_t, slot_c, rows_t, rows_c,
             wv_v, bv_v, out_v, sem_t, sem_c):
    c = lax.axis_index("c")
    s = lax.axis_index("s")
    wid = s * NC + c
    base = wid * BPW

    pltpu.sync_copy(wv_hbm, wv_v)
    pltpu.sync_copy(bv_hbm, bv_v)
    wv = wv_v[...]
    bv = bv_v[...]
    iota = lax.iota(jnp.int32, L)

    for ch in range(NCHUNK):
        cbase = base + ch * CHUNK
        pltpu.sync_copy(tgt_hbm.at[pl.ds(cbase, CHUNK)], idx_t)
        pltpu.sync_copy(ctx_hbm.at[pl.ds(cbase, CHUNK)], idx_c)

        @plsc.parallel_loop(0, CGROUPS)
        def _slots(j):
            sl = pl.ds(j * L, L)
            slot_t[sl] = idx_t[sl] >> 1
            slot_c[sl] = idx_c[sl] >> 1

        cp_t = pltpu.async_copy(table_hbm.at[slot_t], rows_t, sem_t)
        cp_c = pltpu.async_copy(table_hbm.at[slot_c], rows_c, sem_c)
        cp_t.wait()
        cp_c.wait()

        @plsc.parallel_loop(0, CGROUPS)
        def _group(g):
            sl = pl.ds(g * L, L)
            kt = (idx_t[sl] & 1) * EMB_DIM
            kc = (idx_c[sl] & 1) * EMB_DIM
            rows = g * L + iota
            acc = jnp.zeros((L,), jnp.float32)
            for d in range(EMB_DIM):
                tv = plsc.load_gather(rows_t, [rows, kt + d])
                cv = plsc.load_gather(rows_c, [rows, kc + d])
                acc = acc + tv * cv
            x = acc * wv + bv
            y = 1.0 / (1.0 + jnp.exp(-x))
            out_v[pl.ds(ch * CHUNK + g * L, L)] = y

    pltpu.sync_copy(out_v, out_hbm.at[pl.ds(base, BPW)])


@jax.jit
def kernel(pair, table, dense_w, dense_b):
    tgt = pair[:, 0]
    ctx = pair[:, 1]
    wv = jnp.broadcast_to(dense_w.reshape(()), (L,))
    bv = jnp.broadcast_to(dense_b.reshape(()), (L,))
    table2 = table.reshape(NSLOTS, SLOT_W)

    mesh = plsc.VectorSubcoreMesh(core_axis_name="c", subcore_axis_name="s")
    run = pl.kernel(
        _sc_body,
        out_type=jax.ShapeDtypeStruct((BATCH,), jnp.float32),
        mesh=mesh,
        compiler_params=pltpu.CompilerParams(
            needs_layout_passes=False, use_tc_tiling_on_sc=False),
        scratch_types=[
            pltpu.VMEM((CHUNK,), jnp.int32),
            pltpu.VMEM((CHUNK,), jnp.int32),
            pltpu.VMEM((CHUNK,), jnp.int32),
            pltpu.VMEM((CHUNK,), jnp.int32),
            pltpu.VMEM((CHUNK, SLOT_W), jnp.float32),
            pltpu.VMEM((CHUNK, SLOT_W), jnp.float32),
            pltpu.VMEM((L,), jnp.float32),
            pltpu.VMEM((L,), jnp.float32),
            pltpu.VMEM((BPW,), jnp.float32),
            pltpu.SemaphoreType.DMA,
            pltpu.SemaphoreType.DMA,
        ],
    )
    out = run(tgt, ctx, wv, bv, table2)
    return out.reshape(BATCH, 1)


# speedup vs baseline: 1.6348x; 1.6348x over previous
import jax
import jax.numpy as jnp
from jax import lax
from jax.experimental import pallas as pl
from jax.experimental.pallas import tpu as pltpu
from jax.experimental.pallas import tpu_sc as plsc

NR_NODES = 1000000
EMB_DIM = 64
BATCH = 16384

NC = 2
NS = 16
L = 16
NW = NC * NS
BPW = BATCH // NW
CHUNK = 128
NCHUNK = BPW // CHUNK
CGROUPS = CHUNK // L


def _sc_body(tgt_hbm, ctx_hbm, wv_hbm, bv_hbm, table_hbm, out_hbm,
             idx_t, idx_c, rows_t, rows_c, wv_v, bv_v, out_v,
             sem_t0, sem_t1, sem_c0, sem_c1):
    c = lax.axis_index("c")
    s = lax.axis_index("s")
    wid = s * NC + c
    base = wid * BPW

    pltpu.sync_copy(tgt_hbm.at[pl.ds(base, BPW)], idx_t)
    pltpu.sync_copy(ctx_hbm.at[pl.ds(base, BPW)], idx_c)
    pltpu.sync_copy(wv_hbm, wv_v)
    pltpu.sync_copy(bv_hbm, bv_v)
    wv = wv_v[...]
    bv = bv_v[...]
    iota = lax.iota(jnp.int32, L)

    for ch in range(NCHUNK):
        @plsc.parallel_loop(0, CGROUPS, step=2)
        def _fire(g):
            goff = ch * CHUNK + g * L
            vt0 = idx_t[pl.ds(goff, L)]
            vc0 = idx_c[pl.ds(goff, L)]
            vt1 = idx_t[pl.ds(goff + L, L)]
            vc1 = idx_c[pl.ds(goff + L, L)]
            for j in range(L):
                i0 = g * L + j
                i1 = (g + 1) * L + j
                pltpu.async_copy(table_hbm.at[vt0[j]], rows_t.at[i0], sem_t0)
                pltpu.async_copy(table_hbm.at[vc0[j]], rows_c.at[i0], sem_c0)
                pltpu.async_copy(table_hbm.at[vt1[j]], rows_t.at[i1], sem_t1)
                pltpu.async_copy(table_hbm.at[vc1[j]], rows_c.at[i1], sem_c1)

        half = pl.ds(0, CHUNK // 2)
        dummy = table_hbm.at[pl.ds(0, CHUNK // 2)]
        pltpu.make_async_copy(dummy, rows_t.at[half], sem_t0).wait()
        pltpu.make_async_copy(dummy, rows_t.at[half], sem_t1).wait()
        pltpu.make_async_copy(dummy, rows_c.at[half], sem_c0).wait()
        pltpu.make_async_copy(dummy, rows_c.at[half], sem_c1).wait()

        @plsc.parallel_loop(0, CGROUPS)
        def _group(g):
            rows = g * L + iota
            acc = jnp.zeros((L,), jnp.float32)
            for d in range(EMB_DIM):
                col = jnp.full((L,), d, jnp.int32)
                tv = plsc.load_gather(rows_t, [rows, col])
                cv = plsc.load_gather(rows_c, [rows, col])
                acc = acc + tv * cv
            x = acc * wv + bv
            y = 1.0 / (1.0 + jnp.exp(-x))
            out_v[pl.ds(ch * CHUNK + g * L, L)] = y

    pltpu.sync_copy(out_v, out_hbm.at[pl.ds(base, BPW)])


@jax.jit
def kernel(pair, table, dense_w, dense_b):
    tgt = pair[:, 0]
    ctx = pair[:, 1]
    wv = jnp.broadcast_to(dense_w.reshape(()), (L,))
    bv = jnp.broadcast_to(dense_b.reshape(()), (L,))

    mesh = plsc.VectorSubcoreMesh(core_axis_name="c", subcore_axis_name="s")
    run = pl.kernel(
        _sc_body,
        out_type=jax.ShapeDtypeStruct((BATCH,), jnp.float32),
        mesh=mesh,
        compiler_params=pltpu.CompilerParams(
            needs_layout_passes=False, use_tc_tiling_on_sc=True),
        scratch_types=[
            pltpu.VMEM((BPW,), jnp.int32),
            pltpu.VMEM((BPW,), jnp.int32),
            pltpu.VMEM((CHUNK, EMB_DIM), jnp.float32),
            pltpu.VMEM((CHUNK, EMB_DIM), jnp.float32),
            pltpu.VMEM((L,), jnp.float32),
            pltpu.VMEM((L,), jnp.float32),
            pltpu.VMEM((BPW,), jnp.float32),
            pltpu.SemaphoreType.DMA,
            pltpu.SemaphoreType.DMA,
            pltpu.SemaphoreType.DMA,
            pltpu.SemaphoreType.DMA,
        ],
    )
    out = run(tgt, ctx, wv, bv, table)
    return out.reshape(BATCH, 1)


# speedup vs baseline: 1.6437x; 1.0055x over previous
import jax
import jax.numpy as jnp
from jax import lax
from jax.experimental import pallas as pl
from jax.experimental.pallas import tpu as pltpu
from jax.experimental.pallas import tpu_sc as plsc

NR_NODES = 1000000
EMB_DIM = 64
BATCH = 16384

NC = 2
NS = 16
L = 16
NW = NC * NS
BPW = BATCH // NW
CHUNK = 128
NCHUNK = BPW // CHUNK
CGROUPS = CHUNK // L


def _sc_body(tgt_hbm, ctx_hbm, wv_hbm, bv_hbm, table_hbm, out_hbm,
             idx_t, idx_c, rows_t0, rows_c0, rows_t1, rows_c1,
             wv_v, bv_v, out_v, sem_t0, sem_t1, sem_c0, sem_c1):
    c = lax.axis_index("c")
    s = lax.axis_index("s")
    wid = s * NC + c
    base = wid * BPW

    pltpu.sync_copy(tgt_hbm.at[pl.ds(base, BPW)], idx_t)
    pltpu.sync_copy(ctx_hbm.at[pl.ds(base, BPW)], idx_c)
    pltpu.sync_copy(wv_hbm, wv_v)
    pltpu.sync_copy(bv_hbm, bv_v)
    wv = wv_v[...]
    bv = bv_v[...]
    iota = lax.iota(jnp.int32, L)

    bufs = ((rows_t0, rows_c0), (rows_t1, rows_c1))
    sems = ((sem_t0, sem_c0), (sem_t1, sem_c1))

    def fire(ch):
        rt, rc = bufs[ch % 2]
        st, sc2 = sems[ch % 2]

        @plsc.parallel_loop(0, CGROUPS)
        def _fire(g):
            goff = ch * CHUNK + g * L
            vt = idx_t[pl.ds(goff, L)]
            vc = idx_c[pl.ds(goff, L)]
            for j in range(L):
                i = g * L + j
                pltpu.async_copy(table_hbm.at[vt[j]], rt.at[i], st)
                pltpu.async_copy(table_hbm.at[vc[j]], rc.at[i], sc2)

    dummy = table_hbm.at[pl.ds(0, CHUNK)]
    fire(0)
    for ch in range(NCHUNK):
        if ch + 1 < NCHUNK:
            fire(ch + 1)
        rt, rc = bufs[ch % 2]
        st, sc2 = sems[ch % 2]
        pltpu.make_async_copy(dummy, rt, st).wait()
        pltpu.make_async_copy(dummy, rc, sc2).wait()

        @plsc.parallel_loop(0, CGROUPS)
        def _group(g):
            rows = g * L + iota
            acc = jnp.zeros((L,), jnp.float32)
            for d in range(EMB_DIM):
                col = jnp.full((L,), d, jnp.int32)
                tv = plsc.load_gather(rt, [rows, col])
                cv = plsc.load_gather(rc, [rows, col])
                acc = acc + tv * cv
            x = acc * wv + bv
            y = 1.0 / (1.0 + jnp.exp(-x))
            out_v[pl.ds(ch * CHUNK + g * L, L)] = y

    pltpu.sync_copy(out_v, out_hbm.at[pl.ds(base, BPW)])


@jax.jit
def kernel(pair, table, dense_w, dense_b):
    tgt = pair[:, 0]
    ctx = pair[:, 1]
    wv = jnp.broadcast_to(dense_w.reshape(()), (L,))
    bv = jnp.broadcast_to(dense_b.reshape(()), (L,))

    mesh = plsc.VectorSubcoreMesh(core_axis_name="c", subcore_axis_name="s")
    run = pl.kernel(
        _sc_body,
        out_type=jax.ShapeDtypeStruct((BATCH,), jnp.float32),
        mesh=mesh,
        compiler_params=pltpu.CompilerParams(
            needs_layout_passes=False, use_tc_tiling_on_sc=True),
        scratch_types=[
            pltpu.VMEM((BPW,), jnp.int32),
            pltpu.VMEM((BPW,), jnp.int32),
            pltpu.VMEM((CHUNK, EMB_DIM), jnp.float32),
            pltpu.VMEM((CHUNK, EMB_DIM), jnp.float32),
            pltpu.VMEM((CHUNK, EMB_DIM), jnp.float32),
            pltpu.VMEM((CHUNK, EMB_DIM), jnp.float32),
            pltpu.VMEM((L,), jnp.float32),
            pltpu.VMEM((L,), jnp.float32),
            pltpu.VMEM((BPW,), jnp.float32),
            pltpu.SemaphoreType.DMA,
            pltpu.SemaphoreType.DMA,
            pltpu.SemaphoreType.DMA,
            pltpu.SemaphoreType.DMA,
        ],
    )
    out = run(tgt, ctx, wv, bv, table)
    return out.reshape(BATCH, 1)
